# baseline (speedup 1.0000x reference)
import jax
import jax.numpy as jnp
from jax.experimental import pallas as pl
from jax.experimental.pallas import tpu as pltpu

_BLOCK_T = 512
_NBUF = 4


def _in_copy(x_hbm, in_ref, in_sem, step, slot):
    return pltpu.make_async_copy(
        x_hbm.at[pl.ds(step * _BLOCK_T, _BLOCK_T), :],
        in_ref.at[slot],
        in_sem.at[slot],
    )


def _out_copy(o_ref, o_hbm, o_sem, step, slot):
    return pltpu.make_async_copy(
        o_ref.at[slot],
        o_hbm.at[pl.ds(step * _BLOCK_T, _BLOCK_T), :],
        o_sem.at[slot],
    )


def _router_kernel(x_hbm, wt_ref, b_ref, mask_hbm, probs_hbm,
                   in_ref, m_ref, p_ref, in_sem, m_sem, p_sem):
    i = pl.program_id(0)
    nsteps = pl.num_programs(0)

    @pl.when(i == 0)
    def _prologue():
        for s in range(_NBUF):
            _in_copy(x_hbm, in_ref, in_sem, s, s).start()

    slot = jax.lax.rem(i, _NBUF)
    _in_copy(x_hbm, in_ref, in_sem, i, slot).wait()

    x = in_ref[slot]
    logits = jnp.dot(x, wt_ref[...], preferred_element_type=jnp.float32)
    logits = logits + b_ref[...]

    @pl.when(i >= _NBUF)
    def _drain():
        _out_copy(m_ref, mask_hbm, m_sem, i - _NBUF, slot).wait()
        _out_copy(p_ref, probs_hbm, p_sem, i - _NBUF, slot).wait()

    m = jnp.max(logits, axis=-1, keepdims=True)
    e = jnp.exp(logits - m)
    p_ref[slot] = e / jnp.sum(e, axis=-1, keepdims=True)
    E = logits.shape[-1]
    iota = jax.lax.broadcasted_iota(jnp.int32, logits.shape, 1)
    first = jnp.min(jnp.where(logits == m, iota, E), axis=-1, keepdims=True)
    m_ref[slot] = (iota == first).astype(jnp.float32)

    _out_copy(m_ref, mask_hbm, m_sem, i, slot).start()
    _out_copy(p_ref, probs_hbm, p_sem, i, slot).start()

    @pl.when(i + _NBUF < nsteps)
    def _prefetch():
        _in_copy(x_hbm, in_ref, in_sem, i + _NBUF, slot).start()

    @pl.when(i == nsteps - 1)
    def _epilogue():
        for s in range(_NBUF):
            _out_copy(m_ref, mask_hbm, m_sem, 0, s).wait()
            _out_copy(p_ref, probs_hbm, p_sem, 0, s).wait()


def kernel(x, W, b):
    T, D = x.shape
    E = W.shape[0]
    wt = W.T
    b2 = b.reshape(1, E)
    grid = (T // _BLOCK_T,)
    mask, probs = pl.pallas_call(
        _router_kernel,
        grid=grid,
        in_specs=[
            pl.BlockSpec(memory_space=pltpu.HBM),
            pl.BlockSpec((D, E), lambda i: (0, 0)),
            pl.BlockSpec((1, E), lambda i: (0, 0)),
        ],
        out_specs=[
            pl.BlockSpec(memory_space=pltpu.HBM),
            pl.BlockSpec(memory_space=pltpu.HBM),
        ],
        out_shape=[
            jax.ShapeDtypeStruct((T, E), jnp.float32),
            jax.ShapeDtypeStruct((T, E), jnp.float32),
        ],
        scratch_shapes=[
            pltpu.VMEM((_NBUF, _BLOCK_T, D), jnp.float32),
            pltpu.VMEM((_NBUF, _BLOCK_T, E), jnp.float32),
            pltpu.VMEM((_NBUF, _BLOCK_T, E), jnp.float32),
            pltpu.SemaphoreType.DMA((_NBUF,)),
            pltpu.SemaphoreType.DMA((_NBUF,)),
            pltpu.SemaphoreType.DMA((_NBUF,)),
        ],
        compiler_params=pltpu.CompilerParams(
            dimension_semantics=("arbitrary",),
        ),
    )(x, wt, b2)
    return (mask, probs)
